# Initial kernel scaffold; baseline (speedup 1.0000x reference)
import jax
import jax.numpy as jnp
from jax.experimental import pallas as pl


def kernel(means3d, opacities, scales, rotations, features):
    raise NotImplementedError("write your pallas kernel here")


# speedup vs baseline: 10.6225x; 10.6225x over previous
import jax
import jax.numpy as jnp
from jax import lax
from jax.experimental import pallas as pl
from jax.experimental.pallas import tpu as pltpu
from jax.experimental.pallas import tpu_sc as plsc

_H, _W, _D = 64, 64, 8
_N = 1024
_F = 32
_X0 = -16.0
_Y0 = -16.0
_Z0 = -2.0
_NC = 2
_NS = 16
_NW = _NC * _NS
_P = 16


def _prep_body(means_ref, opac_ref, scales_ref, rot_ref, params_ref):
    q = rot_ref[...]
    s = scales_ref[...]
    m = means_ref[...]
    nrm = jnp.sqrt(jnp.sum(q * q, axis=1, keepdims=True)) + 1e-8
    q = q / nrm
    w = q[:, 0:1]
    x = q[:, 1:2]
    y = q[:, 2:3]
    z = q[:, 3:4]
    r00 = 1.0 - 2.0 * (y * y + z * z)
    r01 = 2.0 * (x * y - w * z)
    r02 = 2.0 * (x * z + w * y)
    r10 = 2.0 * (x * y + w * z)
    r11 = 1.0 - 2.0 * (x * x + z * z)
    r12 = 2.0 * (y * z - w * x)
    r20 = 2.0 * (x * z - w * y)
    r21 = 2.0 * (y * z + w * x)
    r22 = 1.0 - 2.0 * (x * x + y * y)
    s0 = s[:, 0:1]
    s1 = s[:, 1:2]
    s2 = s[:, 2:3]
    m00, m01, m02 = r00 * s0, r01 * s1, r02 * s2
    m10, m11, m12 = r10 * s0, r11 * s1, r12 * s2
    m20, m21, m22 = r20 * s0, r21 * s1, r22 * s2
    eps = 1e-6
    c00 = m00 * m00 + m01 * m01 + m02 * m02 + eps
    c11 = m10 * m10 + m11 * m11 + m12 * m12 + eps
    c22 = m20 * m20 + m21 * m21 + m22 * m22 + eps
    c01 = m00 * m10 + m01 * m11 + m02 * m12
    c02 = m00 * m20 + m01 * m21 + m02 * m22
    c12 = m10 * m20 + m11 * m21 + m12 * m22
    cof00 = c11 * c22 - c12 * c12
    cof01 = c02 * c12 - c01 * c22
    cof02 = c01 * c12 - c11 * c02
    det = c00 * cof00 + c01 * cof01 + c02 * cof02
    inv_det = 1.0 / det
    a00 = cof00 * inv_det
    a11 = (c00 * c22 - c02 * c02) * inv_det
    a22 = (c00 * c11 - c01 * c01) * inv_det
    a01 = cof01 * inv_det
    a02 = cof02 * inv_det
    a12 = (c01 * c02 - c00 * c12) * inv_det
    t3x = 3.0 * jnp.sqrt(jnp.maximum(c00, 1e-8))
    t3y = 3.0 * jnp.sqrt(jnp.maximum(c11, 1e-8))
    t3z = 3.0 * jnp.sqrt(jnp.maximum(c22, 1e-8))
    op = opac_ref[...]
    pad = jnp.zeros_like(op)
    params_ref[...] = jnp.concatenate(
        [m[:, 0:1], m[:, 1:2], m[:, 2:3], t3x, t3y, t3z,
         a00, a11, a22, 2.0 * a01, 2.0 * a02, 2.0 * a12,
         op, pad, pad, pad], axis=1)


_C6 = (2.18784062e-04, 1.23874111e-03, 9.68464805e-03, 5.54803926e-02,
       2.40230494e-01, 6.93146937e-01, 1.00000000e+00)


def _floor_i(x):
    return (x + 2048.0).astype(jnp.int32) - 2048


def _ceil_i(x):
    return 2048 - (2048.0 - x).astype(jnp.int32)


def _splat_body(params_hbm, feats_hbm, out_hbm, pp, pf, acc):
    wid = lax.axis_index("s") * _NC + lax.axis_index("c")
    pltpu.sync_copy(params_hbm, pp)
    pltpu.sync_copy(feats_hbm, pf)
    lane = lax.iota(jnp.int32, 16)
    lane_f = lane.astype(jnp.float32)
    zc = (lane_f + 0.5) * 0.5 + _Z0
    lane_ok = lane < _D
    zero16 = jnp.zeros((16,), jnp.float32)

    for rp in range(_H // _NW):
        r = wid + rp * _NW
        cr = (r.astype(jnp.float32) + 0.5) * 0.5 + _X0

        def zbody(i, c):
            acc[pl.ds(i * 16, 16)] = zero16
            return c
        lax.fori_loop(0, _W * _D * _F // 16, zbody, 0)

        def gbody(g, c):
            prow = pp[pl.ds(g * _P, 16)]
            mx = prow[0]
            t3x = prow[3]
            dx = cr - mx

            @pl.when(jnp.abs(dx) <= t3x)
            def _():
                my = prow[1]
                mz = prow[2]
                t3y = prow[4]
                t3z = prow[5]
                a00 = prow[6]
                a11 = prow[7]
                a22 = prow[8]
                a01x2 = prow[9]
                a02x2 = prow[10]
                a12x2 = prow[11]
                op = prow[12]
                dz = zc - mz
                zgate = jnp.where((jnp.abs(dz) <= t3z) & lane_ok, op, 0.0)
                zq = a22 * (dz * dz)
                fv0 = pf[pl.ds(g * _F, 16)]
                fv1 = pf[pl.ds(g * _F + 16, 16)]
                j0 = jnp.maximum(_ceil_i((my - t3y - _Y0) * 2.0 - 0.5), 0)
                j1 = jnp.minimum(_floor_i((my + t3y - _Y0) * 2.0 - 0.5), _W - 1)
                qx = a00 * dx * dx
                lx = a02x2 * dx
                lyc = a01x2 * dx

                def jbody(j, c2):
                    dy = (j.astype(jnp.float32) + 0.5) * 0.5 + _Y0 - my
                    qs = qx + lyc * dy + a11 * dy * dy
                    ls = lx + a12x2 * dy
                    t = jnp.maximum((qs + ls * dz + zq) * -0.7213475204444817,
                                    -126.0)
                    n = (t + 2048.0).astype(jnp.int32) - 2048
                    f = t - n.astype(jnp.float32)
                    p = _C6[0]
                    for cc in _C6[1:]:
                        p = p * f + cc
                    scale = lax.bitcast_convert_type((n + 127) << 23,
                                                     jnp.float32)
                    dens = zgate * (p * scale)
                    cb = j * (_D * _F)
                    for zz in range(_D):
                        d = dens[zz]
                        plsc.addupdate(acc.at[pl.ds(cb + zz * _F, 16)], d * fv0)
                        plsc.addupdate(acc.at[pl.ds(cb + zz * _F + 16, 16)], d * fv1)
                    return c2
                lax.fori_loop(j0, j1 + 1, jbody, 0)
            return c
        lax.fori_loop(0, _N, gbody, 0)
        pltpu.sync_copy(acc, out_hbm.at[r])


def kernel(means3d, opacities, scales, rotations, features):
    params = pl.pallas_call(
        _prep_body,
        out_shape=jax.ShapeDtypeStruct((_N, _P), jnp.float32),
    )(means3d, opacities, scales, rotations)
    mesh = plsc.VectorSubcoreMesh(core_axis_name="c", subcore_axis_name="s")
    splat = pl.kernel(
        _splat_body,
        out_type=jax.ShapeDtypeStruct((_H, _W * _D * _F), jnp.float32),
        mesh=mesh,
        scratch_types=[
            pltpu.VMEM((_N * _P,), jnp.float32),
            pltpu.VMEM((_N * _F,), jnp.float32),
            pltpu.VMEM((_W * _D * _F,), jnp.float32),
        ],
    )
    out = splat(params.reshape(-1), features.reshape(-1))
    return out.reshape(_H, _W, _D, _F)
